# Initial kernel scaffold; baseline (speedup 1.0000x reference)
import jax
import jax.numpy as jnp
from jax.experimental import pallas as pl


def kernel(pos, batch):
    raise NotImplementedError("write your pallas kernel here")


# speedup vs baseline: 7.6936x; 7.6936x over previous
import functools

import jax
import jax.numpy as jnp
from jax import lax
from jax.experimental import pallas as pl
from jax.experimental.pallas import tpu as pltpu
from jax.experimental.pallas import tpu_sc as plsc

_B = 16
_P = 4096
_S = 1024
_L = 16
_CH = _P // _L


def _fps_body(x_hbm, y_hbm, z_hbm, out_hbm, x_v, y_v, z_v, dist_v, idx_v):
    c = lax.axis_index("c")
    s = lax.axis_index("s")
    b = s

    @pl.when(c == 0)
    def _():
        lanes = lax.iota(jnp.int32, _L)
        pltpu.sync_copy(x_hbm.at[pl.ds(b * _P, _P)], x_v)
        pltpu.sync_copy(y_hbm.at[pl.ds(b * _P, _P)], y_v)
        pltpu.sync_copy(z_hbm.at[pl.ds(b * _P, _P)], z_v)

        def init_chunk(i, _):
            dist_v[pl.ds(i * _L, _L)] = jnp.full((_L,), jnp.inf, jnp.float32)
            return 0

        lax.fori_loop(jnp.int32(0), jnp.int32(_CH), init_chunk, 0)

        plsc.store_scatter(idx_v, [jnp.zeros((_L,), jnp.int32)],
                           jnp.full((_L,), b * _P, jnp.int32),
                           mask=lanes == 0)

        def outer(i, sel):
            selv = jnp.full((_L,), sel, jnp.int32)
            sx = plsc.load_gather(x_v, [selv])
            sy = plsc.load_gather(y_v, [selv])
            sz = plsc.load_gather(z_v, [selv])

            def chunk(k, carry):
                rmax, ridx = carry
                sl = pl.ds(k * _L, _L)
                dx = x_v[sl] - sx
                dy = y_v[sl] - sy
                dz = z_v[sl] - sz
                d = dx * dx + dy * dy + dz * dz
                dmin = jnp.minimum(dist_v[sl], d)
                dist_v[sl] = dmin
                pred = dmin > rmax
                rmax = jnp.where(pred, dmin, rmax)
                ridx = jnp.where(pred, jnp.full((_L,), k, jnp.int32), ridx)
                return rmax, ridx

            rmax, ridx = lax.fori_loop(
                jnp.int32(0), jnp.int32(_CH), chunk,
                (jnp.full((_L,), -1.0, jnp.float32),
                 jnp.zeros((_L,), jnp.int32)))

            m = jnp.max(rmax)
            lin = ridx * _L + lanes
            cand = jnp.where(rmax == m, lin, jnp.int32(2 ** 30))
            nsel = jnp.min(cand)
            plsc.store_scatter(idx_v, [jnp.full((_L,), i, jnp.int32)],
                               jnp.full((_L,), b * _P + nsel, jnp.int32),
                               mask=lanes == 0)
            return nsel

        lax.fori_loop(jnp.int32(1), jnp.int32(_S), outer, jnp.int32(0))
        pltpu.sync_copy(idx_v, out_hbm.at[b])


_fps_kernel = functools.partial(
    pl.kernel,
    out_type=jax.ShapeDtypeStruct((_B, _S), jnp.int32),
    mesh=plsc.VectorSubcoreMesh(core_axis_name="c", subcore_axis_name="s",
                                num_cores=2, num_subcores=16),
    compiler_params=pltpu.CompilerParams(needs_layout_passes=False),
    scratch_types=[
        pltpu.VMEM((_P,), jnp.float32),
        pltpu.VMEM((_P,), jnp.float32),
        pltpu.VMEM((_P,), jnp.float32),
        pltpu.VMEM((_P,), jnp.float32),
        pltpu.VMEM((_S,), jnp.int32),
    ],
)(_fps_body)


def kernel(pos, batch):
    del batch
    x = pos[:, 0]
    y = pos[:, 1]
    z = pos[:, 2]
    idx = _fps_kernel(x, y, z)
    return idx.reshape(-1).astype(jnp.int64)


# speedup vs baseline: 7.6974x; 1.0005x over previous
import functools

import jax
import jax.numpy as jnp
from jax import lax
from jax.experimental import pallas as pl
from jax.experimental.pallas import tpu as pltpu
from jax.experimental.pallas import tpu_sc as plsc

_B = 16
_P = 4096
_S = 1024
_L = 16
_CH = _P // _L


def _fps_body(x_hbm, y_hbm, z_hbm, out_hbm, x_v, y_v, z_v, dist_v, idx_v):
    c = lax.axis_index("c")
    s = lax.axis_index("s")
    b = s

    @pl.when(c == 0)
    def _():
        lanes = lax.iota(jnp.int32, _L)
        pltpu.sync_copy(x_hbm.at[pl.ds(b * _P, _P)], x_v)
        pltpu.sync_copy(y_hbm.at[pl.ds(b * _P, _P)], y_v)
        pltpu.sync_copy(z_hbm.at[pl.ds(b * _P, _P)], z_v)

        def init_chunk(i, _):
            for u in range(8):
                dist_v[pl.ds(i * (8 * _L) + u * _L, _L)] = jnp.full(
                    (_L,), jnp.inf, jnp.float32)
            return 0

        lax.fori_loop(jnp.int32(0), jnp.int32(_CH // 8), init_chunk, 0)

        plsc.store_scatter(idx_v, [jnp.zeros((_L,), jnp.int32)],
                           jnp.full((_L,), b * _P, jnp.int32),
                           mask=lanes == 0)

        def outer(i, sel):
            selv = jnp.full((_L,), sel, jnp.int32)
            sx = plsc.load_gather(x_v, [selv])
            sy = plsc.load_gather(y_v, [selv])
            sz = plsc.load_gather(z_v, [selv])

            def chunk(k8, carry):
                rmax, ridx = carry
                for u in range(8):
                    k = k8 * 8 + u
                    sl = pl.ds(k * _L, _L)
                    dx = x_v[sl] - sx
                    dy = y_v[sl] - sy
                    dz = z_v[sl] - sz
                    d = dx * dx + dy * dy + dz * dz
                    dmin = jnp.minimum(dist_v[sl], d)
                    dist_v[sl] = dmin
                    pred = dmin > rmax
                    rmax = jnp.where(pred, dmin, rmax)
                    ridx = jnp.where(pred, jnp.full((_L,), k, jnp.int32),
                                     ridx)
                return rmax, ridx

            rmax, ridx = lax.fori_loop(
                jnp.int32(0), jnp.int32(_CH // 8), chunk,
                (jnp.full((_L,), -1.0, jnp.float32),
                 jnp.zeros((_L,), jnp.int32)))

            m = jnp.max(rmax)
            lin = ridx * _L + lanes
            cand = jnp.where(rmax == m, lin, jnp.int32(2 ** 30))
            nsel = jnp.min(cand)
            plsc.store_scatter(idx_v, [jnp.full((_L,), i, jnp.int32)],
                               jnp.full((_L,), b * _P + nsel, jnp.int32),
                               mask=lanes == 0)
            return nsel

        lax.fori_loop(jnp.int32(1), jnp.int32(_S), outer, jnp.int32(0))
        pltpu.sync_copy(idx_v, out_hbm.at[b])


_fps_kernel = functools.partial(
    pl.kernel,
    out_type=jax.ShapeDtypeStruct((_B, _S), jnp.int32),
    mesh=plsc.VectorSubcoreMesh(core_axis_name="c", subcore_axis_name="s",
                                num_cores=2, num_subcores=16),
    compiler_params=pltpu.CompilerParams(needs_layout_passes=False),
    scratch_types=[
        pltpu.VMEM((_P,), jnp.float32),
        pltpu.VMEM((_P,), jnp.float32),
        pltpu.VMEM((_P,), jnp.float32),
        pltpu.VMEM((_P,), jnp.float32),
        pltpu.VMEM((_S,), jnp.int32),
    ],
)(_fps_body)


def kernel(pos, batch):
    del batch
    x = pos[:, 0]
    y = pos[:, 1]
    z = pos[:, 2]
    idx = _fps_kernel(x, y, z)
    return idx.reshape(-1).astype(jnp.int64)
